# baseline (speedup 1.0000x reference)
import functools

import jax
import jax.numpy as jnp
from jax import lax
from jax.experimental import pallas as pl
from jax.experimental.pallas import tpu as pltpu

_INV_TEMP = 0.5


def _cos_sum_kernel(c1_ref, c2_ref, o_ref, acc_ref, *, inv_temp, n_valid,
                    tile_m, need_mask):
    k = pl.program_id(0)

    @pl.when(k == 0)
    def _():
        acc_ref[...] = jnp.zeros_like(acc_ref)

    c1 = c1_ref[...].astype(jnp.float32)
    c2 = c2_ref[...].astype(jnp.float32)

    s = jnp.sum(c1 * c2, axis=-1, keepdims=True)
    a = jnp.sum(c1 * c1, axis=-1, keepdims=True)
    b = jnp.sum(c2 * c2, axis=-1, keepdims=True)

    eps2 = jnp.float32(1e-24)
    cos = s * lax.rsqrt(jnp.maximum(a, eps2) * jnp.maximum(b, eps2))

    if need_mask:
        rows = k * tile_m + lax.broadcasted_iota(jnp.int32, cos.shape, 0)
        cos = jnp.where(rows < n_valid, cos, jnp.float32(0.0))

    acc_ref[...] += jnp.sum(cos)

    @pl.when(k == pl.num_programs(0) - 1)
    def _():
        o_ref[...] = acc_ref[...] * jnp.float32(inv_temp)


def kernel(concept1, concept2):
    bs, d = concept1.shape
    n_class, d2 = concept2.shape
    assert d == d2, "embedding dims must match"

    n_diag = min(bs, n_class)
    c1 = concept1[:n_diag]
    c2 = concept2[:n_diag]

    itemsize = jnp.dtype(concept1.dtype).itemsize
    tile_budget_rows = max((28 << 20) // (4 * d * itemsize), 8)
    tile_m = min(1792, tile_budget_rows)
    if tile_m >= n_diag:
        tile_m = n_diag
    else:
        tile_m = max(tile_m // 8 * 8, 8)
    n_steps = pl.cdiv(n_diag, tile_m)
    need_mask = (n_diag % tile_m) != 0

    out = pl.pallas_call(
        functools.partial(
            _cos_sum_kernel,
            inv_temp=_INV_TEMP,
            n_valid=n_diag,
            tile_m=tile_m,
            need_mask=need_mask,
        ),
        out_shape=jax.ShapeDtypeStruct((1, 1), jnp.float32),
        grid=(n_steps,),
        in_specs=[
            pl.BlockSpec((tile_m, d), lambda k: (k, 0)),
            pl.BlockSpec((tile_m, d), lambda k: (k, 0)),
        ],
        out_specs=pl.BlockSpec((1, 1), lambda k: (0, 0)),
        scratch_shapes=[pltpu.VMEM((1, 1), jnp.float32)],
        compiler_params=pltpu.CompilerParams(
            dimension_semantics=("arbitrary",),
        ),
    )(c1, c2)
    return out[0, 0]


# speedup vs baseline: 1.0724x; 1.0724x over previous
import functools

import jax
import jax.numpy as jnp
from jax import lax
from jax.experimental import pallas as pl
from jax.experimental.pallas import tpu as pltpu

_INV_TEMP = 0.5


def _cos_sum_kernel(c1_ref, c2_ref, o_ref, acc_ref, *, inv_temp, n_valid,
                    tile_m, need_mask):
    k = pl.program_id(0)

    @pl.when(k == 0)
    def _():
        acc_ref[...] = jnp.zeros_like(acc_ref)

    c1 = c1_ref[...].astype(jnp.float32)
    c2 = c2_ref[...].astype(jnp.float32)

    s = jnp.sum(c1 * c2, axis=-1, keepdims=True)
    a = jnp.sum(c1 * c1, axis=-1, keepdims=True)
    b = jnp.sum(c2 * c2, axis=-1, keepdims=True)

    eps2 = jnp.float32(1e-24)
    cos = s * lax.rsqrt(jnp.maximum(a, eps2) * jnp.maximum(b, eps2))

    if need_mask:
        rows = k * tile_m + lax.broadcasted_iota(jnp.int32, cos.shape, 0)
        cos = jnp.where(rows < n_valid, cos, jnp.float32(0.0))

    acc_ref[...] += jnp.sum(cos)

    @pl.when(k == pl.num_programs(0) - 1)
    def _():
        o_ref[...] = acc_ref[...] * jnp.float32(inv_temp)


def kernel(concept1, concept2):
    bs, d = concept1.shape
    n_class, d2 = concept2.shape
    assert d == d2, "embedding dims must match"

    n_diag = min(bs, n_class)
    c1 = concept1[:n_diag]
    c2 = concept2[:n_diag]

    itemsize = jnp.dtype(concept1.dtype).itemsize
    tile_budget_rows = max((28 << 20) // (4 * d * itemsize), 8)
    tile_m = min(1280, tile_budget_rows)
    if tile_m >= n_diag:
        tile_m = n_diag
    else:
        tile_m = max(tile_m // 8 * 8, 8)
    n_steps = pl.cdiv(n_diag, tile_m)
    need_mask = (n_diag % tile_m) != 0

    out = pl.pallas_call(
        functools.partial(
            _cos_sum_kernel,
            inv_temp=_INV_TEMP,
            n_valid=n_diag,
            tile_m=tile_m,
            need_mask=need_mask,
        ),
        out_shape=jax.ShapeDtypeStruct((1, 1), jnp.float32),
        grid=(n_steps,),
        in_specs=[
            pl.BlockSpec((tile_m, d), lambda k: (k, 0)),
            pl.BlockSpec((tile_m, d), lambda k: (k, 0)),
        ],
        out_specs=pl.BlockSpec((1, 1), lambda k: (0, 0)),
        scratch_shapes=[pltpu.VMEM((1, 1), jnp.float32)],
        compiler_params=pltpu.CompilerParams(
            dimension_semantics=("arbitrary",),
        ),
    )(c1, c2)
    return out[0, 0]
